# baseline (speedup 1.0000x reference)
import functools

import jax
import jax.numpy as jnp
from jax import lax
from jax.experimental import pallas as pl
from jax.experimental.pallas import tpu as pltpu
from jax.experimental.pallas import tpu_sc as plsc

VOCAB = 1000000
EMBED = 32
B, L = 4096, 200
N = B * L

NC, NS, LANES = 2, 16, 16
NW = NC * NS
RPB = 128
N_LT = L // 8
N_BT = B // RPB
N_ET = EMBED // 8
BB = RPB // LANES


def _sc_embed(tok5, val5, tt, vt):
    mesh = plsc.VectorSubcoreMesh(core_axis_name="c", subcore_axis_name="s")

    @functools.partial(
        pl.kernel,
        mesh=mesh,
        compiler_params=pltpu.CompilerParams(
            use_tc_tiling_on_sc=False, needs_layout_passes=False),
        out_type=jax.ShapeDtypeStruct((L, N_ET, N_BT, 8, RPB), jnp.float32),
        scratch_types=[
            pltpu.VMEM((8, RPB), jnp.int32),
            pltpu.VMEM((8, RPB), jnp.float32),
            pltpu.VMEM((RPB, EMBED), jnp.float32),
            pltpu.VMEM((RPB, EMBED), jnp.float32),
            pltpu.VMEM((EMBED, RPB), jnp.float32),
            pltpu.SemaphoreType.DMA,
        ],
    )
    def k(tok_hbm, val_hbm, tt_hbm, vt_hbm, out_hbm, idx_v, vals_v, trows, vrows, outT, sem):
        cid = lax.axis_index("c")
        sid = lax.axis_index("s")
        w = sid * NC + cid

        iota = lax.iota(jnp.int32, LANES)

        def lt_body(lt, carry):
            pltpu.sync_copy(tok_hbm.at[lt, w], idx_v)
            pltpu.sync_copy(val_hbm.at[lt, w], vals_v)

            def li_body(li, c2):
                cp1 = pltpu.async_copy(tt_hbm.at[idx_v.at[li]], trows, sem)
                cp2 = pltpu.async_copy(vt_hbm.at[idx_v.at[li]], vrows, sem)
                cp1.wait()
                cp2.wait()
                for bb in range(BB):
                    tokv = idx_v[li, pl.ds(bb * LANES, LANES)]
                    valv = vals_v[li, pl.ds(bb * LANES, LANES)]
                    nz = tokv != 0
                    m = jnp.where(nz, 1.0, 0.0)
                    wv = jnp.where(nz & (valv == valv), valv, 0.0)
                    row_idx = iota + (bb * LANES)
                    for e in range(EMBED):
                        col_idx = jnp.full((LANES,), e, jnp.int32)
                        t = plsc.load_gather(trows, [row_idx, col_idx])
                        v = plsc.load_gather(vrows, [row_idx, col_idx])
                        outT[e, pl.ds(bb * LANES, LANES)] = t * m + v * wv
                l = lt * 8 + li
                for et in range(N_ET):
                    pltpu.sync_copy(outT.at[pl.ds(et * 8, 8)], out_hbm.at[l, et, w])
                return c2

            lax.fori_loop(0, 8, li_body, 0)
            return carry

        lax.fori_loop(0, N_LT, lt_body, 0)

    return k(tok5, val5, tt, vt)


def kernel(tokens, values, token_table, value_table):
    tok5 = tokens.T.reshape(N_LT, 8, N_BT, RPB).transpose(0, 2, 1, 3)
    val5 = values.T.reshape(N_LT, 8, N_BT, RPB).transpose(0, 2, 1, 3)
    out5 = _sc_embed(tok5, val5, token_table, value_table)
    return out5.transpose(2, 4, 0, 1, 3).reshape(B, L, EMBED)


# speedup vs baseline: 1.1383x; 1.1383x over previous
import functools

import jax
import jax.numpy as jnp
from jax import lax
from jax.experimental import pallas as pl
from jax.experimental.pallas import tpu as pltpu
from jax.experimental.pallas import tpu_sc as plsc

VOCAB = 1000000
EMBED = 32
B, L = 4096, 200
N = B * L

NC, NS, LANES = 2, 16, 16
NW = NC * NS
RPB = 128
N_LT = L // 8
N_BT = B // RPB
N_ET = EMBED // 8
BB = RPB // LANES
NU = L


def _sc_embed(tok5, val5, tt, vt):
    mesh = plsc.VectorSubcoreMesh(core_axis_name="c", subcore_axis_name="s")

    @functools.partial(
        pl.kernel,
        mesh=mesh,
        compiler_params=pltpu.CompilerParams(
            use_tc_tiling_on_sc=False, needs_layout_passes=False),
        out_type=jax.ShapeDtypeStruct((L, N_ET, N_BT, 8, RPB), jnp.float32),
        scratch_types=[
            pltpu.VMEM((2, 8, RPB), jnp.int32),
            pltpu.VMEM((2, 8, RPB), jnp.float32),
            pltpu.VMEM((2, RPB, EMBED), jnp.float32),
            pltpu.VMEM((2, RPB, EMBED), jnp.float32),
            pltpu.VMEM((2, EMBED, RPB), jnp.float32),
            pltpu.SemaphoreType.DMA((2,)),
            pltpu.SemaphoreType.DMA((2,)),
            pltpu.SemaphoreType.DMA,
        ],
    )
    def k(tok_hbm, val_hbm, tt_hbm, vt_hbm, out_hbm,
          idx_v, vals_v, trows, vrows, outT, gsem, osem, ssem):
        cid = lax.axis_index("c")
        sid = lax.axis_index("s")
        w = sid * NC + cid

        iota = lax.iota(jnp.int32, LANES)

        def fire_gather(u):
            bf = u % 2
            lt2 = (u // 8) % 2
            li = u % 8
            pltpu.async_copy(tt_hbm.at[idx_v.at[lt2, li]], trows.at[bf], gsem.at[bf])
            pltpu.async_copy(vt_hbm.at[idx_v.at[lt2, li]], vrows.at[bf], gsem.at[bf])

        def wait_gather(u):
            bf = u % 2
            pltpu.make_async_copy(tt_hbm.at[pl.ds(0, RPB)], trows.at[bf], gsem.at[bf]).wait()
            pltpu.make_async_copy(vt_hbm.at[pl.ds(0, RPB)], vrows.at[bf], gsem.at[bf]).wait()

        def drain_out(u):
            bf = u % 2
            l = u // 8 * 8 + u % 8
            for et in range(N_ET):
                pltpu.make_async_copy(
                    outT.at[bf, pl.ds(et * 8, 8)], out_hbm.at[u, et, w], osem.at[bf]).wait()

        pltpu.sync_copy(tok_hbm.at[0, w], idx_v.at[0])
        pltpu.sync_copy(val_hbm.at[0, w], vals_v.at[0])
        fire_gather(0)

        def unit_body(u, carry):
            lt = u // 8
            li = u % 8
            lt2 = lt % 2
            bf = u % 2

            @pl.when(jnp.logical_and(li == 0, lt < N_LT - 1))
            def _():
                pltpu.async_copy(tok_hbm.at[lt + 1, w], idx_v.at[(lt + 1) % 2], ssem)
                pltpu.async_copy(val_hbm.at[lt + 1, w], vals_v.at[(lt + 1) % 2], ssem)

            @pl.when(jnp.logical_and(li == 7, lt < N_LT - 1))
            def _():
                pltpu.make_async_copy(tok_hbm.at[0, w], idx_v.at[0], ssem).wait()
                pltpu.make_async_copy(val_hbm.at[0, w], vals_v.at[0], ssem).wait()

            @pl.when(u < NU - 1)
            def _():
                fire_gather(u + 1)

            wait_gather(u)

            @pl.when(u >= 2)
            def _():
                drain_out(u - 2)

            for bb in range(BB):
                tokv = idx_v[lt2, li, pl.ds(bb * LANES, LANES)]
                valv = vals_v[lt2, li, pl.ds(bb * LANES, LANES)]
                nz = tokv != 0
                m = jnp.where(nz, 1.0, 0.0)
                wv = jnp.where(nz & (valv == valv), valv, 0.0)
                row_idx = iota + (bb * LANES)
                for e in range(EMBED):
                    col_idx = jnp.full((LANES,), e, jnp.int32)
                    t = plsc.load_gather(trows.at[bf], [row_idx, col_idx])
                    v = plsc.load_gather(vrows.at[bf], [row_idx, col_idx])
                    outT[bf, e, pl.ds(bb * LANES, LANES)] = t * m + v * wv

            for et in range(N_ET):
                pltpu.async_copy(
                    outT.at[bf, pl.ds(et * 8, 8)], out_hbm.at[u, et, w], osem.at[bf])
            return carry

        lax.fori_loop(0, NU, unit_body, 0)
        drain_out(NU - 2)
        drain_out(NU - 1)

    return k(tok5, val5, tt, vt)


def kernel(tokens, values, token_table, value_table):
    tok5 = tokens.T.reshape(N_LT, 8, N_BT, RPB).transpose(0, 2, 1, 3)
    val5 = values.T.reshape(N_LT, 8, N_BT, RPB).transpose(0, 2, 1, 3)
    out5 = _sc_embed(tok5, val5, token_table, value_table)
    return out5.transpose(2, 4, 0, 1, 3).reshape(B, L, EMBED)
